# Initial kernel scaffold; baseline (speedup 1.0000x reference)
import jax
import jax.numpy as jnp
from jax.experimental import pallas as pl


def kernel(src, ind, log_amp, phase, locs, grid, mag_coef, epicenter_spatial_coef, depth_spatial_coef, bias, k):
    raise NotImplementedError("write your pallas kernel here")


# speedup vs baseline: 1.3885x; 1.3885x over previous
import functools

import jax
import jax.numpy as jnp
from jax import lax
from jax.experimental import pallas as pl
from jax.experimental.pallas import tpu as pltpu
from jax.experimental.pallas import tpu_sc as plsc

_B = 2048
_NSTA = 250
_NG = 20000
_K = 8
_GP = 20480
_BT = 128
_BT2 = 256
_DP = 512
_CP = 256
_NW = 32
_QW = _B // _NW
_CQ = 8
_NCH = _QW // _CQ


def _knn_body(src_ref, gt_ref, out_ref, d2_ref):
    s = src_ref[...] * jnp.float32(1e-3)
    gt = gt_ref[...] * jnp.float32(1e-3)
    g2 = jnp.sum(gt * gt, axis=0, keepdims=True)
    d2 = g2 - 2.0 * jnp.dot(s, gt, preferred_element_type=jnp.float32)
    col = lax.broadcasted_iota(jnp.int32, (_BT, _GP), 1)
    d2_ref[...] = jnp.where(col < _NG, d2, jnp.inf)
    for i in range(_K):
        d2c = d2_ref[...]
        m = jnp.min(d2c, axis=1, keepdims=True)
        cand = jnp.where(d2c == m, col, jnp.int32(2**30))
        idx = jnp.min(cand, axis=1, keepdims=True)
        out_ref[:, i:i + 1] = idx
        d2_ref[...] = jnp.where(col == idx, jnp.inf, d2c)


_knn = pl.pallas_call(
    _knn_body,
    grid=(_B // _BT,),
    in_specs=[
        pl.BlockSpec((_BT, 8), lambda i: (i, 0)),
        pl.BlockSpec((8, _GP), lambda i: (0, 0)),
    ],
    out_specs=pl.BlockSpec((_BT, _K), lambda i: (i, 0)),
    out_shape=jax.ShapeDtypeStruct((_B, _K), jnp.int32),
    scratch_shapes=[pltpu.VMEM((_BT, _GP), jnp.float32)],
)


def _sc_gather_body(bias_hbm, inds_hbm, cols_hbm, out_hbm,
                    idx_v, cols_v, buf, acc_v, sem):
    cid = lax.axis_index("c")
    sid = lax.axis_index("s")
    wid = sid * 2 + cid
    qbase = wid * _QW
    pltpu.sync_copy(inds_hbm.at[pl.ds(qbase * _K, _QW * _K)], idx_v)
    pltpu.sync_copy(cols_hbm, cols_v)
    colv = [cols_v[pl.ds(j * 16, 16)] for j in range(_CP // 16)]

    def chunk(ch, carry):
        idx_slice = idx_v.at[pl.ds(ch * (_CQ * _K), _CQ * _K)]
        pltpu.async_copy(bias_hbm.at[idx_slice], buf, sem).wait()
        for q in range(_CQ):
            rowv = [jnp.full((16,), q * _K + kk, jnp.int32)
                    for kk in range(_K)]
            arow = ch * _CQ + q
            for j in range(_CP // 16):
                a = plsc.load_gather(buf, [rowv[0], colv[j]])
                for kk in range(1, _K):
                    a = a + plsc.load_gather(buf, [rowv[kk], colv[j]])
                acc_v[pl.ds(arow * _CP + j * 16, 16)] = a
        return carry

    lax.fori_loop(0, _NCH, chunk, 0)
    pltpu.sync_copy(acc_v, out_hbm.at[pl.ds(qbase * _CP, _QW * _CP)])


@functools.cache
def _get_sc_gather():
    return functools.partial(
        pl.kernel,
        out_type=jax.ShapeDtypeStruct((_B * _CP,), jnp.float32),
        mesh=plsc.VectorSubcoreMesh(core_axis_name="c",
                                    subcore_axis_name="s"),
        scratch_types=[
            pltpu.VMEM((_QW * _K,), jnp.int32),
            pltpu.VMEM((_CP,), jnp.int32),
            pltpu.VMEM((_CQ * _K, _DP), jnp.float32),
            pltpu.VMEM((_QW * _CP,), jnp.float32),
            pltpu.SemaphoreType.DMA,
        ],
        compiler_params=pltpu.CompilerParams(
            use_tc_tiling_on_sc=False,
            needs_layout_passes=False,
        ),
    )(_sc_gather_body)


def _combine_body(params_ref, src_ref, la_ref, acc_ref, lt_ref, ind_ref,
                  out_ref):
    e = params_ref[0]
    dc = params_ref[1]
    mc = params_ref[2]
    iota_s = lax.broadcasted_iota(jnp.int32, (_CP, _CP), 0)
    oh = (iota_s == ind_ref[...]).astype(jnp.float32)
    lperm = jnp.dot(lt_ref[...], oh,
                    preferred_element_type=jnp.float32)
    lx = lperm[0:1, :]
    ly = lperm[1:2, :]
    lz = lperm[2:3, :]
    sx = src_ref[:, 0:1]
    sy = src_ref[:, 1:2]
    sz = src_ref[:, 2:3]
    dh = jnp.sqrt((sx - lx) ** 2 + (sy - ly) ** 2)
    pwz = jnp.log10(dh + 1.0)
    pwd = jnp.log10(jnp.abs(sz - lz) + 1.0)
    bias_sel = acc_ref[...] * jnp.float32(1.0 / _K)
    mag = (la_ref[...] - e * pwz[:, :_NSTA] - dc * pwd[:, :_NSTA]
           - bias_sel[:, :_NSTA]) / jnp.maximum(mc, jnp.float32(1e-12))
    out_ref[...] = mag


_combine = pl.pallas_call(
    _combine_body,
    grid=(_B // _BT2,),
    in_specs=[
        pl.BlockSpec(memory_space=pltpu.SMEM),
        pl.BlockSpec((_BT2, 8), lambda i: (i, 0)),
        pl.BlockSpec((_BT2, _NSTA), lambda i: (i, 0)),
        pl.BlockSpec((_BT2, _CP), lambda i: (i, 0)),
        pl.BlockSpec((8, _CP), lambda i: (0, 0)),
        pl.BlockSpec((1, _CP), lambda i: (0, 0)),
    ],
    out_specs=pl.BlockSpec((_BT2, _NSTA), lambda i: (i, 0)),
    out_shape=jax.ShapeDtypeStruct((_B, _NSTA), jnp.float32),
)


def kernel(src, ind, log_amp, phase, locs, grid, mag_coef,
           epicenter_spatial_coef, depth_spatial_coef, bias, k):
    del k
    src = src.astype(jnp.float32)
    src_pad = jnp.pad(src, ((0, 0), (0, 5)))
    gt_pad = jnp.pad(grid.T, ((0, 5), (0, _GP - _NG)))
    bias_pad = jnp.pad(bias.reshape(_NG, 2 * _NSTA),
                       ((0, 0), (0, _DP - 2 * _NSTA)))
    ind32 = ind.astype(jnp.int32)
    phase32 = jnp.asarray(phase, jnp.int32)
    cols = jnp.pad(ind32 * 2 + phase32, ((0, _CP - _NSTA),))
    ind_row = jnp.pad(ind32, ((0, _CP - _NSTA),)).reshape(1, _CP)
    locst_pad = jnp.pad(locs.T, ((0, 5), (0, _CP - _NSTA)))
    params = jnp.stack([
        epicenter_spatial_coef[phase],
        depth_spatial_coef[phase],
        mag_coef[phase],
    ]).astype(jnp.float32)

    inds = _knn(src_pad, gt_pad)
    acc = _get_sc_gather()(bias_pad, inds.reshape(-1), cols)
    acc = acc.reshape(_B, _CP)
    return _combine(params, src_pad, log_amp.astype(jnp.float32), acc,
                    locst_pad, ind_row)
